# Initial kernel scaffold; baseline (speedup 1.0000x reference)
import jax
import jax.numpy as jnp
from jax.experimental import pallas as pl


def kernel(object_logits, object_boxes, lane_logits, lane_polylines, od_boxes, od_labels, od_valid, lane_targets, lane_valid):
    raise NotImplementedError("write your pallas kernel here")


# speedup vs baseline: 20.2305x; 20.2305x over previous
import functools

import jax
import jax.numpy as jnp
from jax import lax
from jax.experimental import pallas as pl
from jax.experimental.pallas import tpu as pltpu
from jax.experimental.pallas import tpu_sc as plsc

F32 = jnp.float32
I32 = jnp.int32

B = 8
Q, C, T = 900, 10, 128
LQ, PF, TL = 100, 60, 32
QP = 912
LQP = 112
BIG = 1.0e30
BIGC = 3.0e30
OKTHR = 1.0e29
IBIG = 2 ** 30

DET_W = T * QP
LANE_W = TL * LQP


def _stage1_body(olT_ref, obT_ref, tb_ref, lab_ref, val_ref, ll_ref, lpT_ref,
                 ltg_ref, lval_ref,
                 costd_ref, cmind_ref, cargd_ref, costl_ref, cminl_ref,
                 cargl_ref, bce1_ref, bce2_ref, nv_ref, lnv_ref):
    olT = olT_ref[0]
    obT = obT_ref[0]
    tb = tb_ref[0]
    lab = lab_ref[0]
    val = val_ref[0]
    probsT = 1.0 / (1.0 + jnp.exp(-olT))

    gath = jnp.zeros((T, Q), F32)
    for c in range(C):
        gath = jnp.where(lab == c, probsT[c:c + 1, :], gath)
    class_cost = 1.0 - gath

    def bdiff(d):
        return jnp.abs(obT[d:d + 1, :] - tb[:, d:d + 1])

    center_cost = bdiff(0) + bdiff(1) + bdiff(2)
    size_cost = (jnp.abs(jnp.abs(obT[3:4, :]) - jnp.abs(tb[:, 3:4]))
                 + jnp.abs(jnp.abs(obT[4:5, :]) - jnp.abs(tb[:, 4:5]))
                 + jnp.abs(jnp.abs(obT[5:6, :]) - jnp.abs(tb[:, 5:6])))
    dy = obT[6:7, :] - tb[:, 6:7]
    yaw_cost = jnp.abs(jnp.arctan2(jnp.sin(dy), jnp.cos(dy)))
    vel_cost = bdiff(7) + bdiff(8)
    cost = (2.0 * class_cost + 2.0 * center_cost + 1.0 * size_cost
            + 0.5 * yaw_cost + 0.5 * vel_cost)
    cost = jnp.where(val != 0.0, cost, BIG)
    costd_ref[0, :, 0:Q] = cost
    costd_ref[0, :, Q:QP] = jnp.full((T, QP - Q), BIG, F32)

    m_t = jnp.min(cost, axis=1, keepdims=True)
    qio = lax.broadcasted_iota(I32, (T, Q), 1)
    a_t = jnp.min(jnp.where(cost == m_t, qio, IBIG), axis=1, keepdims=True)
    cmind_ref[0] = m_t
    cargd_ref[0] = a_t

    lpT = lpT_ref[0]
    ltg = ltg_ref[0]
    lval = lval_ref[0]
    lacc = jnp.zeros((TL, LQ), F32)
    for j in range(PF):
        lacc = lacc + jnp.abs(lpT[j:j + 1, :] - ltg[:, j:j + 1])
    lcost = (lacc / F32(PF)) * 4.0
    lcost = jnp.where(lval != 0.0, lcost, BIG)
    costl_ref[0, :, 0:LQ] = lcost
    costl_ref[0, :, LQ:LQP] = jnp.full((TL, LQP - LQ), BIG, F32)

    lm_t = jnp.min(lcost, axis=1, keepdims=True)
    lqio = lax.broadcasted_iota(I32, (TL, LQ), 1)
    la_t = jnp.min(jnp.where(lcost == lm_t, lqio, IBIG), axis=1, keepdims=True)
    cminl_ref[0] = lm_t
    cargl_ref[0] = la_t

    bce1_ref[0] = jnp.full((1, 1), jnp.sum(jnp.maximum(olT, 0.0)
                                           + jnp.log1p(jnp.exp(-jnp.abs(olT)))))
    ll = ll_ref[0]
    bce2_ref[0] = jnp.full((1, 1), jnp.sum(jnp.maximum(ll, 0.0)
                                           + jnp.log1p(jnp.exp(-jnp.abs(ll)))))
    nv_ref[0] = jnp.full((1, 1), jnp.sum(val))
    lnv_ref[0] = jnp.full((1, 1), jnp.sum(lval))


def _stage1(olT, obT, tb, lab3, val3, ll3, lpT, ltg3, lval3):
    spec = lambda *shape: pl.BlockSpec((1,) + shape, lambda b: (b, 0, 0))
    return pl.pallas_call(
        _stage1_body,
        grid=(B,),
        in_specs=[
            spec(C, Q), spec(9, Q), spec(T, 9), spec(T, 1), spec(T, 1),
            spec(1, LQ), spec(PF, LQ), spec(TL, PF), spec(TL, 1),
        ],
        out_specs=[
            spec(T, QP), spec(T, 1), spec(T, 1),
            spec(TL, LQP), spec(TL, 1), spec(TL, 1),
            spec(1, 1), spec(1, 1), spec(1, 1), spec(1, 1),
        ],
        out_shape=[
            jax.ShapeDtypeStruct((B, T, QP), F32),
            jax.ShapeDtypeStruct((B, T, 1), F32),
            jax.ShapeDtypeStruct((B, T, 1), I32),
            jax.ShapeDtypeStruct((B, TL, LQP), F32),
            jax.ShapeDtypeStruct((B, TL, 1), F32),
            jax.ShapeDtypeStruct((B, TL, 1), I32),
            jax.ShapeDtypeStruct((B, 1, 1), F32),
            jax.ShapeDtypeStruct((B, 1, 1), F32),
            jax.ShapeDtypeStruct((B, 1, 1), F32),
            jax.ShapeDtypeStruct((B, 1, 1), F32),
        ],
    )(olT, obT, tb, lab3, val3, ll3, lpT, ltg3, lval3)


def _sc_scatter1(ref, idx, value, lane0):
    plsc.store_scatter(ref, [jnp.full((16,), idx, I32)],
                       jnp.full((16,), value, ref.dtype), mask=lane0)


def _greedy(nt, qp, cost_v, cmin_v, carg_v, rpen_v, mr_v, mc_v, mo_v):
    nchunk_t = nt // 16
    nchunk_q = qp // 16
    lanes = lax.iota(I32, 16)
    lane0 = lanes == 0

    for ch in range(nchunk_q):
        rpen_v[pl.ds(ch * 16, 16)] = jnp.zeros((16,), F32)

    def step(k, _):
        acc = jnp.full((16,), F32(3.5e30), F32)
        for j in range(nchunk_t):
            acc = jnp.minimum(acc, cmin_v[pl.ds(j * 16, 16)])
        m = jnp.min(acc)
        kacc = jnp.full((16,), IBIG, I32)
        for j in range(nchunk_t):
            cm = cmin_v[pl.ds(j * 16, 16)]
            ca = carg_v[pl.ds(j * 16, 16)]
            key = ca * 128 + (lanes + j * 16)
            kacc = jnp.minimum(kacc, jnp.where(cm == m, key, IBIG))
        kmin = jnp.min(kacc)
        col = jnp.bitwise_and(kmin, 127)
        r = jnp.right_shift(kmin, 7)
        okv = (m < OKTHR).astype(I32)

        _sc_scatter1(mr_v, k, r, lane0)
        _sc_scatter1(mc_v, k, col, lane0)
        _sc_scatter1(mo_v, k, okv, lane0)
        _sc_scatter1(rpen_v, r, BIG, lane0)
        _sc_scatter1(cmin_v, col, BIGC, lane0)
        _sc_scatter1(carg_v, col, jnp.int32(-1), lane0)

        def find_j():
            jacc = jnp.full((16,), IBIG, I32)
            for j in range(nchunk_t):
                ca = carg_v[pl.ds(j * 16, 16)]
                jacc = jnp.minimum(jacc,
                                   jnp.where(ca == r, lanes + j * 16, IBIG))
            return jnp.min(jacc)

        def recompute(jc):
            base = jc * qp

            def chunk(ch, carry):
                bmin, brow = carry
                v = (cost_v[pl.ds(base + ch * 16, 16)]
                     + rpen_v[pl.ds(ch * 16, 16)])
                rows = lanes + ch * 16
                brow = jnp.where(v < bmin, rows, brow)
                return jnp.minimum(bmin, v), brow

            bmin, brow = lax.fori_loop(
                0, nchunk_q, chunk,
                (jnp.full((16,), F32(3.5e30), F32), jnp.zeros((16,), I32)))
            m2 = jnp.min(bmin)
            r2 = jnp.min(jnp.where(bmin == m2, brow, IBIG))
            _sc_scatter1(cmin_v, jc, m2, lane0)
            _sc_scatter1(carg_v, jc, r2, lane0)
            return find_j()

        lax.while_loop(lambda jc: jc < IBIG, recompute, find_j())
        return 0

    lax.fori_loop(0, nt, step, 0)


def _sc_match(costd_f, cmind_f, cargd_f, costl_f, cminl_f, cargl_f):
    mesh = plsc.VectorSubcoreMesh(core_axis_name="c", subcore_axis_name="s")

    @functools.partial(
        pl.kernel,
        mesh=mesh,
        compiler_params=pltpu.CompilerParams(needs_layout_passes=False),
        out_type=[
            jax.ShapeDtypeStruct((B * T,), I32),
            jax.ShapeDtypeStruct((B * T,), I32),
            jax.ShapeDtypeStruct((B * T,), I32),
            jax.ShapeDtypeStruct((B * TL,), I32),
            jax.ShapeDtypeStruct((B * TL,), I32),
            jax.ShapeDtypeStruct((B * TL,), I32),
        ],
        scratch_types=[
            pltpu.VMEM((DET_W,), F32),
            pltpu.VMEM((T,), F32),
            pltpu.VMEM((T,), I32),
            pltpu.VMEM((QP,), F32),
            pltpu.VMEM((T,), I32),
            pltpu.VMEM((T,), I32),
            pltpu.VMEM((T,), I32),
        ],
    )
    def k(costd, cmind, cargd, costl, cminl, cargl,
          outdr, outdc, outdo, outlr, outlc, outlo,
          cost_v, cmin_v, carg_v, rpen_v, mr_v, mc_v, mo_v):
        wid = lax.axis_index("s") * 2 + lax.axis_index("c")

        @pl.when(wid < B)
        def _det():
            b = wid
            pltpu.sync_copy(costd.at[pl.ds(b * DET_W, DET_W)], cost_v)
            pltpu.sync_copy(cmind.at[pl.ds(b * T, T)], cmin_v)
            pltpu.sync_copy(cargd.at[pl.ds(b * T, T)], carg_v)
            _greedy(T, QP, cost_v, cmin_v, carg_v, rpen_v, mr_v, mc_v, mo_v)
            pltpu.sync_copy(mr_v, outdr.at[pl.ds(b * T, T)])
            pltpu.sync_copy(mc_v, outdc.at[pl.ds(b * T, T)])
            pltpu.sync_copy(mo_v, outdo.at[pl.ds(b * T, T)])

        @pl.when((wid >= B) & (wid < 2 * B))
        def _lane():
            b = wid - B
            pltpu.sync_copy(costl.at[pl.ds(b * LANE_W, LANE_W)],
                            cost_v.at[pl.ds(0, LANE_W)])
            pltpu.sync_copy(cminl.at[pl.ds(b * TL, TL)],
                            cmin_v.at[pl.ds(0, TL)])
            pltpu.sync_copy(cargl.at[pl.ds(b * TL, TL)],
                            carg_v.at[pl.ds(0, TL)])
            _greedy(TL, LQP, cost_v, cmin_v, carg_v, rpen_v, mr_v, mc_v, mo_v)
            pltpu.sync_copy(mr_v.at[pl.ds(0, TL)], outlr.at[pl.ds(b * TL, TL)])
            pltpu.sync_copy(mc_v.at[pl.ds(0, TL)], outlc.at[pl.ds(b * TL, TL)])
            pltpu.sync_copy(mo_v.at[pl.ds(0, TL)], outlo.at[pl.ds(b * TL, TL)])

    return k(costd_f, cmind_f, cargd_f, costl_f, cminl_f, cargl_f)


def _stage3_body(mrd_ref, mcd_ref, mod_ref, mrl_ref, mcl_ref, mol_ref,
                 ob_ref, tb_ref, ol_ref, lab_ref, ll_ref, lp_ref, ltg_ref,
                 bce1_ref, bce2_ref, nv_ref, lnv_ref, out_ref):
    hi = jax.lax.Precision.HIGHEST
    tot_box = F32(0.0)
    tot_pos = F32(0.0)
    tot_lshape = F32(0.0)
    tot_lpos = F32(0.0)
    for b in range(B):
        mr = mrd_ref[b]
        mc = mcd_ref[b]
        mo = mod_ref[b].astype(F32)
        qio = lax.broadcasted_iota(I32, (T, Q), 1)
        ohr = (mr == qio).astype(F32)
        tio = lax.broadcasted_iota(I32, (T, T), 1)
        ohc = (mc == tio).astype(F32)
        pred = jnp.dot(ohr, ob_ref[b], precision=hi)
        tgt = jnp.dot(ohc, tb_ref[b], precision=hi)
        d = pred - tgt
        ad = jnp.abs(d)
        sl1 = jnp.where(ad < 1.0, 0.5 * d * d, ad - 0.5)
        tot_box = tot_box + jnp.sum(sl1 * mo)
        glog = jnp.dot(ohr, ol_ref[b], precision=hi)
        cio = lax.broadcasted_iota(I32, (T, C), 1)
        lab_oh = (lab_ref[b] == cio).astype(F32)
        glab = jnp.dot(ohc, lab_oh, precision=hi)
        tot_pos = tot_pos + jnp.sum(glog * glab * mo)

        lmr = mrl_ref[b]
        lmc = mcl_ref[b]
        lmo = mol_ref[b].astype(F32)
        lqio = lax.broadcasted_iota(I32, (TL, LQ), 1)
        ohlr = (lmr == lqio).astype(F32)
        ltio = lax.broadcasted_iota(I32, (TL, TL), 1)
        ohlc = (lmc == ltio).astype(F32)
        predl = jnp.dot(ohlr, lp_ref[b], precision=hi)
        tgtl = jnp.dot(ohlc, ltg_ref[b], precision=hi)
        dl = predl - tgtl
        adl = jnp.abs(dl)
        sl1l = jnp.where(adl < 1.0, 0.5 * dl * dl, adl - 0.5)
        tot_lshape = tot_lshape + jnp.sum(sl1l * lmo)
        tot_lpos = tot_lpos + jnp.sum(ohlr * ll_ref[b] * lmo)

    bce1 = jnp.sum(bce1_ref[...])
    bce2 = jnp.sum(bce2_ref[...])
    norm = jnp.maximum(jnp.sum(nv_ref[...]), 1.0)
    lnorm = jnp.maximum(jnp.sum(lnv_ref[...]), 1.0)
    oio = lax.broadcasted_iota(I32, (1, 6), 1)
    out_ref[...] = ((oio == 1).astype(F32) * ((bce1 - tot_pos) / norm)
                    + (oio == 2).astype(F32) * (tot_box / norm)
                    + (oio == 4).astype(F32) * ((bce2 - tot_lpos) / lnorm)
                    + (oio == 5).astype(F32) * (tot_lshape / lnorm))


def _stage3(mrd, mcd, mod_, mrl, mcl, mol, ob, tb, ol, lab3, ll3, lpf, ltg3,
            bce1, bce2, nv, lnv):
    return pl.pallas_call(
        _stage3_body,
        out_shape=jax.ShapeDtypeStruct((1, 6), F32),
    )(mrd, mcd, mod_, mrl, mcl, mol, ob, tb, ol, lab3, ll3, lpf, ltg3,
      bce1, bce2, nv, lnv)


def kernel(object_logits, object_boxes, lane_logits, lane_polylines,
           od_boxes, od_labels, od_valid, lane_targets, lane_valid):
    ol = object_logits.astype(F32)
    ob = object_boxes.astype(F32)
    ll = lane_logits.astype(F32)
    lp = lane_polylines.astype(F32)
    tb = od_boxes.astype(F32)
    ltg = lane_targets.astype(F32)

    olT = jnp.transpose(ol, (0, 2, 1))
    obT = jnp.transpose(ob, (0, 2, 1))
    lpf = lp.reshape(B, LQ, PF)
    lpT = jnp.transpose(lpf, (0, 2, 1))
    ltg3 = ltg.reshape(B, TL, PF)
    lab3 = od_labels.astype(I32).reshape(B, T, 1)
    val3 = od_valid.astype(F32).reshape(B, T, 1)
    lval3 = lane_valid.astype(F32).reshape(B, TL, 1)
    ll3 = ll.reshape(B, 1, LQ)

    (costd, cmind, cargd, costl, cminl, cargl,
     bce1, bce2, nv, lnv) = _stage1(olT, obT, tb, lab3, val3, ll3, lpT,
                                    ltg3, lval3)

    mrd, mcd, mod_, mrl, mcl, mol = _sc_match(
        costd.reshape(-1), cmind.reshape(-1), cargd.reshape(-1),
        costl.reshape(-1), cminl.reshape(-1), cargl.reshape(-1))

    out = _stage3(mrd.reshape(B, T, 1), mcd.reshape(B, T, 1),
                  mod_.reshape(B, T, 1), mrl.reshape(B, TL, 1),
                  mcl.reshape(B, TL, 1), mol.reshape(B, TL, 1),
                  ob, tb, ol, lab3, ll3, lpf, ltg3, bce1, bce2, nv, lnv)
    return out.reshape(6)


# speedup vs baseline: 115.1756x; 5.6932x over previous
import functools

import jax
import jax.numpy as jnp
from jax import lax
from jax.experimental import pallas as pl
from jax.experimental.pallas import tpu as pltpu
from jax.experimental.pallas import tpu_sc as plsc

F32 = jnp.float32
I32 = jnp.int32

B = 8
Q, C, T = 900, 10, 128
LQ, PF, TL = 100, 60, 32
QP = 912
LQP = 112
BIG = 1.0e30
BIGC = 3.0e30
OKTHR = 1.0e29
IBIG = 2 ** 30

DET_W = T * QP
LANE_W = TL * LQP


def _stage1_body(olT_ref, obT_ref, tb_ref, lab_ref, val_ref, ll_ref, lpT_ref,
                 ltg_ref, lval_ref,
                 costd_ref, cmind_ref, cargd_ref, costl_ref, cminl_ref,
                 cargl_ref, bce1_ref, bce2_ref, nv_ref, lnv_ref):
    olT = olT_ref[0]
    obT = obT_ref[0]
    tb = tb_ref[0]
    lab = lab_ref[0]
    val = val_ref[0]
    probsT = 1.0 / (1.0 + jnp.exp(-olT))

    gath = jnp.zeros((T, Q), F32)
    for c in range(C):
        gath = jnp.where(lab == c, probsT[c:c + 1, :], gath)
    class_cost = 1.0 - gath

    def bdiff(d):
        return jnp.abs(obT[d:d + 1, :] - tb[:, d:d + 1])

    center_cost = bdiff(0) + bdiff(1) + bdiff(2)
    size_cost = (jnp.abs(jnp.abs(obT[3:4, :]) - jnp.abs(tb[:, 3:4]))
                 + jnp.abs(jnp.abs(obT[4:5, :]) - jnp.abs(tb[:, 4:5]))
                 + jnp.abs(jnp.abs(obT[5:6, :]) - jnp.abs(tb[:, 5:6])))
    dy = obT[6:7, :] - tb[:, 6:7]
    yaw_cost = jnp.abs(jnp.arctan2(jnp.sin(dy), jnp.cos(dy)))
    vel_cost = bdiff(7) + bdiff(8)
    cost = (2.0 * class_cost + 2.0 * center_cost + 1.0 * size_cost
            + 0.5 * yaw_cost + 0.5 * vel_cost)
    cost = jnp.where(val != 0.0, cost, BIG)
    costd_ref[0, :, 0:Q] = cost
    costd_ref[0, :, Q:QP] = jnp.full((T, QP - Q), BIG, F32)

    m_t = jnp.min(cost, axis=1, keepdims=True)
    qio = lax.broadcasted_iota(I32, (T, Q), 1)
    a_t = jnp.min(jnp.where(cost == m_t, qio, IBIG), axis=1, keepdims=True)
    cmind_ref[0] = m_t
    cargd_ref[0] = a_t

    lpT = lpT_ref[0]
    ltg = ltg_ref[0]
    lval = lval_ref[0]
    lacc = jnp.zeros((TL, LQ), F32)
    for j in range(PF):
        lacc = lacc + jnp.abs(lpT[j:j + 1, :] - ltg[:, j:j + 1])
    lcost = (lacc / F32(PF)) * 4.0
    lcost = jnp.where(lval != 0.0, lcost, BIG)
    costl_ref[0, :, 0:LQ] = lcost
    costl_ref[0, :, LQ:LQP] = jnp.full((TL, LQP - LQ), BIG, F32)

    lm_t = jnp.min(lcost, axis=1, keepdims=True)
    lqio = lax.broadcasted_iota(I32, (TL, LQ), 1)
    la_t = jnp.min(jnp.where(lcost == lm_t, lqio, IBIG), axis=1, keepdims=True)
    cminl_ref[0] = lm_t
    cargl_ref[0] = la_t

    bce1_ref[0] = jnp.full((1, 1), jnp.sum(jnp.maximum(olT, 0.0)
                                           + jnp.log1p(jnp.exp(-jnp.abs(olT)))))
    ll = ll_ref[0]
    bce2_ref[0] = jnp.full((1, 1), jnp.sum(jnp.maximum(ll, 0.0)
                                           + jnp.log1p(jnp.exp(-jnp.abs(ll)))))
    nv_ref[0] = jnp.full((1, 1), jnp.sum(val))
    lnv_ref[0] = jnp.full((1, 1), jnp.sum(lval))


def _stage1(olT, obT, tb, lab3, val3, ll3, lpT, ltg3, lval3):
    spec = lambda *shape: pl.BlockSpec((1,) + shape, lambda b: (b, 0, 0))
    return pl.pallas_call(
        _stage1_body,
        grid=(B,),
        in_specs=[
            spec(C, Q), spec(9, Q), spec(T, 9), spec(T, 1), spec(T, 1),
            spec(1, LQ), spec(PF, LQ), spec(TL, PF), spec(TL, 1),
        ],
        out_specs=[
            spec(T, QP), spec(T, 1), spec(T, 1),
            spec(TL, LQP), spec(TL, 1), spec(TL, 1),
            spec(1, 1), spec(1, 1), spec(1, 1), spec(1, 1),
        ],
        out_shape=[
            jax.ShapeDtypeStruct((B, T, QP), F32),
            jax.ShapeDtypeStruct((B, T, 1), F32),
            jax.ShapeDtypeStruct((B, T, 1), I32),
            jax.ShapeDtypeStruct((B, TL, LQP), F32),
            jax.ShapeDtypeStruct((B, TL, 1), F32),
            jax.ShapeDtypeStruct((B, TL, 1), I32),
            jax.ShapeDtypeStruct((B, 1, 1), F32),
            jax.ShapeDtypeStruct((B, 1, 1), F32),
            jax.ShapeDtypeStruct((B, 1, 1), F32),
            jax.ShapeDtypeStruct((B, 1, 1), F32),
        ],
    )(olT, obT, tb, lab3, val3, ll3, lpT, ltg3, lval3)


def _sc_scatter1(ref, idx, value, lane0):
    plsc.store_scatter(ref, [jnp.full((16,), idx, I32)],
                       jnp.full((16,), value, ref.dtype), mask=lane0)


def _greedy(nt, qp, cost_v, cmin_v, carg_v, rpen_v, mr_v, mc_v, mo_v):
    nchunk_t = nt // 16
    nchunk_q = qp // 16
    lanes = lax.iota(I32, 16)
    lane0 = lanes == 0

    for ch in range(nchunk_q):
        rpen_v[pl.ds(ch * 16, 16)] = jnp.zeros((16,), F32)

    def compute_m():
        acc = jnp.full((16,), F32(3.5e30), F32)
        for j in range(nchunk_t):
            acc = jnp.minimum(acc, cmin_v[pl.ds(j * 16, 16)])
        return jnp.min(acc)

    def first_stale(m):
        sacc = jnp.full((16,), IBIG, I32)
        for j in range(nchunk_t):
            cm = cmin_v[pl.ds(j * 16, 16)]
            ca = carg_v[pl.ds(j * 16, 16)]
            g = plsc.load_gather(rpen_v, [jnp.maximum(ca, 0)])
            st = (cm == m) & (ca >= 0) & (g > 0.0)
            sacc = jnp.minimum(sacc, jnp.where(st, lanes + j * 16, IBIG))
        return jnp.min(sacc)

    def stale_or_none(m):
        return jnp.where(m < OKTHR, first_stale(m), IBIG)

    def step(k, _):
        def fix_body(carry):
            _, jc = carry
            base = jc * qp

            def chunk(ch, c2):
                bmin, brow = c2
                v = (cost_v[pl.ds(base + ch * 16, 16)]
                     + rpen_v[pl.ds(ch * 16, 16)])
                rows = lanes + ch * 16
                brow = jnp.where(v < bmin, rows, brow)
                return jnp.minimum(bmin, v), brow

            bmin, brow = lax.fori_loop(
                0, nchunk_q, chunk,
                (jnp.full((16,), F32(3.5e30), F32), jnp.zeros((16,), I32)))
            m2 = jnp.min(bmin)
            r2 = jnp.min(jnp.where(bmin == m2, brow, IBIG))
            _sc_scatter1(cmin_v, jc, m2, lane0)
            _sc_scatter1(carg_v, jc, r2, lane0)
            mn = compute_m()
            return mn, stale_or_none(mn)

        m0 = compute_m()
        m, _ = lax.while_loop(lambda c: c[1] < IBIG, fix_body,
                              (m0, stale_or_none(m0)))

        kacc = jnp.full((16,), IBIG, I32)
        for j in range(nchunk_t):
            cm = cmin_v[pl.ds(j * 16, 16)]
            ca = carg_v[pl.ds(j * 16, 16)]
            key = ca * 128 + (lanes + j * 16)
            kacc = jnp.minimum(kacc, jnp.where(cm == m, key, IBIG))
        kmin = jnp.min(kacc)
        col = jnp.bitwise_and(kmin, 127)
        r = jnp.right_shift(kmin, 7)
        ok = m < OKTHR

        _sc_scatter1(mr_v, k, r, lane0)
        _sc_scatter1(mc_v, k, col, lane0)
        _sc_scatter1(mo_v, k, ok.astype(I32), lane0)

        @pl.when(ok)
        def _consume():
            _sc_scatter1(rpen_v, r, BIG, lane0)
            _sc_scatter1(cmin_v, col, BIGC, lane0)
            _sc_scatter1(carg_v, col, jnp.int32(-1), lane0)

        return 0

    lax.fori_loop(0, nt, step, 0)


def _sc_match(costd_f, cmind_f, cargd_f, costl_f, cminl_f, cargl_f):
    mesh = plsc.VectorSubcoreMesh(core_axis_name="c", subcore_axis_name="s")

    @functools.partial(
        pl.kernel,
        mesh=mesh,
        compiler_params=pltpu.CompilerParams(needs_layout_passes=False),
        out_type=[
            jax.ShapeDtypeStruct((B * T,), I32),
            jax.ShapeDtypeStruct((B * T,), I32),
            jax.ShapeDtypeStruct((B * T,), I32),
            jax.ShapeDtypeStruct((B * TL,), I32),
            jax.ShapeDtypeStruct((B * TL,), I32),
            jax.ShapeDtypeStruct((B * TL,), I32),
        ],
        scratch_types=[
            pltpu.VMEM((DET_W,), F32),
            pltpu.VMEM((T,), F32),
            pltpu.VMEM((T,), I32),
            pltpu.VMEM((QP,), F32),
            pltpu.VMEM((T,), I32),
            pltpu.VMEM((T,), I32),
            pltpu.VMEM((T,), I32),
        ],
    )
    def k(costd, cmind, cargd, costl, cminl, cargl,
          outdr, outdc, outdo, outlr, outlc, outlo,
          cost_v, cmin_v, carg_v, rpen_v, mr_v, mc_v, mo_v):
        wid = lax.axis_index("s") * 2 + lax.axis_index("c")

        @pl.when(wid < B)
        def _det():
            b = wid
            pltpu.sync_copy(costd.at[pl.ds(b * DET_W, DET_W)], cost_v)
            pltpu.sync_copy(cmind.at[pl.ds(b * T, T)], cmin_v)
            pltpu.sync_copy(cargd.at[pl.ds(b * T, T)], carg_v)
            _greedy(T, QP, cost_v, cmin_v, carg_v, rpen_v, mr_v, mc_v, mo_v)
            pltpu.sync_copy(mr_v, outdr.at[pl.ds(b * T, T)])
            pltpu.sync_copy(mc_v, outdc.at[pl.ds(b * T, T)])
            pltpu.sync_copy(mo_v, outdo.at[pl.ds(b * T, T)])

        @pl.when((wid >= B) & (wid < 2 * B))
        def _lane():
            b = wid - B
            pltpu.sync_copy(costl.at[pl.ds(b * LANE_W, LANE_W)],
                            cost_v.at[pl.ds(0, LANE_W)])
            pltpu.sync_copy(cminl.at[pl.ds(b * TL, TL)],
                            cmin_v.at[pl.ds(0, TL)])
            pltpu.sync_copy(cargl.at[pl.ds(b * TL, TL)],
                            carg_v.at[pl.ds(0, TL)])
            _greedy(TL, LQP, cost_v, cmin_v, carg_v, rpen_v, mr_v, mc_v, mo_v)
            pltpu.sync_copy(mr_v.at[pl.ds(0, TL)], outlr.at[pl.ds(b * TL, TL)])
            pltpu.sync_copy(mc_v.at[pl.ds(0, TL)], outlc.at[pl.ds(b * TL, TL)])
            pltpu.sync_copy(mo_v.at[pl.ds(0, TL)], outlo.at[pl.ds(b * TL, TL)])

    return k(costd_f, cmind_f, cargd_f, costl_f, cminl_f, cargl_f)


def _stage3_body(mrd_ref, mcd_ref, mod_ref, mrl_ref, mcl_ref, mol_ref,
                 ob_ref, tb_ref, ol_ref, lab_ref, ll_ref, lp_ref, ltg_ref,
                 bce1_ref, bce2_ref, nv_ref, lnv_ref, out_ref):
    hi = jax.lax.Precision.HIGHEST
    tot_box = F32(0.0)
    tot_pos = F32(0.0)
    tot_lshape = F32(0.0)
    tot_lpos = F32(0.0)
    for b in range(B):
        mr = mrd_ref[b]
        mc = mcd_ref[b]
        mo = mod_ref[b].astype(F32)
        qio = lax.broadcasted_iota(I32, (T, Q), 1)
        ohr = (mr == qio).astype(F32)
        tio = lax.broadcasted_iota(I32, (T, T), 1)
        ohc = (mc == tio).astype(F32)
        pred = jnp.dot(ohr, ob_ref[b], precision=hi)
        tgt = jnp.dot(ohc, tb_ref[b], precision=hi)
        d = pred - tgt
        ad = jnp.abs(d)
        sl1 = jnp.where(ad < 1.0, 0.5 * d * d, ad - 0.5)
        tot_box = tot_box + jnp.sum(sl1 * mo)
        glog = jnp.dot(ohr, ol_ref[b], precision=hi)
        cio = lax.broadcasted_iota(I32, (T, C), 1)
        lab_oh = (lab_ref[b] == cio).astype(F32)
        glab = jnp.dot(ohc, lab_oh, precision=hi)
        tot_pos = tot_pos + jnp.sum(glog * glab * mo)

        lmr = mrl_ref[b]
        lmc = mcl_ref[b]
        lmo = mol_ref[b].astype(F32)
        lqio = lax.broadcasted_iota(I32, (TL, LQ), 1)
        ohlr = (lmr == lqio).astype(F32)
        ltio = lax.broadcasted_iota(I32, (TL, TL), 1)
        ohlc = (lmc == ltio).astype(F32)
        predl = jnp.dot(ohlr, lp_ref[b], precision=hi)
        tgtl = jnp.dot(ohlc, ltg_ref[b], precision=hi)
        dl = predl - tgtl
        adl = jnp.abs(dl)
        sl1l = jnp.where(adl < 1.0, 0.5 * dl * dl, adl - 0.5)
        tot_lshape = tot_lshape + jnp.sum(sl1l * lmo)
        tot_lpos = tot_lpos + jnp.sum(ohlr * ll_ref[b] * lmo)

    bce1 = jnp.sum(bce1_ref[...])
    bce2 = jnp.sum(bce2_ref[...])
    norm = jnp.maximum(jnp.sum(nv_ref[...]), 1.0)
    lnorm = jnp.maximum(jnp.sum(lnv_ref[...]), 1.0)
    oio = lax.broadcasted_iota(I32, (1, 6), 1)
    out_ref[...] = ((oio == 1).astype(F32) * ((bce1 - tot_pos) / norm)
                    + (oio == 2).astype(F32) * (tot_box / norm)
                    + (oio == 4).astype(F32) * ((bce2 - tot_lpos) / lnorm)
                    + (oio == 5).astype(F32) * (tot_lshape / lnorm))


def _stage3(mrd, mcd, mod_, mrl, mcl, mol, ob, tb, ol, lab3, ll3, lpf, ltg3,
            bce1, bce2, nv, lnv):
    return pl.pallas_call(
        _stage3_body,
        out_shape=jax.ShapeDtypeStruct((1, 6), F32),
    )(mrd, mcd, mod_, mrl, mcl, mol, ob, tb, ol, lab3, ll3, lpf, ltg3,
      bce1, bce2, nv, lnv)


def kernel(object_logits, object_boxes, lane_logits, lane_polylines,
           od_boxes, od_labels, od_valid, lane_targets, lane_valid):
    ol = object_logits.astype(F32)
    ob = object_boxes.astype(F32)
    ll = lane_logits.astype(F32)
    lp = lane_polylines.astype(F32)
    tb = od_boxes.astype(F32)
    ltg = lane_targets.astype(F32)

    olT = jnp.transpose(ol, (0, 2, 1))
    obT = jnp.transpose(ob, (0, 2, 1))
    lpf = lp.reshape(B, LQ, PF)
    lpT = jnp.transpose(lpf, (0, 2, 1))
    ltg3 = ltg.reshape(B, TL, PF)
    lab3 = od_labels.astype(I32).reshape(B, T, 1)
    val3 = od_valid.astype(F32).reshape(B, T, 1)
    lval3 = lane_valid.astype(F32).reshape(B, TL, 1)
    ll3 = ll.reshape(B, 1, LQ)

    (costd, cmind, cargd, costl, cminl, cargl,
     bce1, bce2, nv, lnv) = _stage1(olT, obT, tb, lab3, val3, ll3, lpT,
                                    ltg3, lval3)

    mrd, mcd, mod_, mrl, mcl, mol = _sc_match(
        costd.reshape(-1), cmind.reshape(-1), cargd.reshape(-1),
        costl.reshape(-1), cminl.reshape(-1), cargl.reshape(-1))

    out = _stage3(mrd.reshape(B, T, 1), mcd.reshape(B, T, 1),
                  mod_.reshape(B, T, 1), mrl.reshape(B, TL, 1),
                  mcl.reshape(B, TL, 1), mol.reshape(B, TL, 1),
                  ob, tb, ol, lab3, ll3, lpf, ltg3, bce1, bce2, nv, lnv)
    return out.reshape(6)
